# baseline (speedup 1.0000x reference)
import functools

import jax
import jax.numpy as jnp
from jax import lax
from jax.experimental import pallas as pl
from jax.experimental.pallas import tpu as pltpu
from jax.experimental.pallas import tpu_sc as plsc

BATCH = 16384
FIELDS = 26
EMBED_DIM = 32
B = BATCH * FIELDS
NC, NS = 2, 16
NW = NC * NS
CHUNK = 128
J = B // (NW * CHUNK)
G = 2
M = J // G
PERW = J * CHUNK
GW = G * CHUNK
PITCH = GW + 1
NROW = 1000000
NBLK = NROW // 4
TBLK = NROW // 128
TSLOT = TBLK // NW
APITCH = 33

_mesh = plsc.VectorSubcoreMesh(core_axis_name="c", subcore_axis_name="s")


def _full(v):
    return jnp.full((16,), v, dtype=jnp.int32)


@functools.partial(
    pl.kernel,
    mesh=_mesh,
    out_type=jax.ShapeDtypeStruct((NBLK, 128), jnp.float32),
    scratch_types=[
        pltpu.VMEM((2, 4, 8, 128), jnp.float32),
        pltpu.VMEM((16, 128), jnp.float32),
        pltpu.VMEM((128 * APITCH,), jnp.float32),
        pltpu.VMEM((32, 128), jnp.float32),
        pltpu.SemaphoreType.DMA,
        pltpu.SemaphoreType.DMA,
        pltpu.SemaphoreType.DMA,
    ],
    compiler_params=pltpu.CompilerParams(
        use_tc_tiling_on_sc=True, needs_layout_passes=False
    ),
)
def _convert_kernel(tabt_hbm, tail_hbm, out_hbm, stg, tailv, tpa, tpb, s0, s1,
                    w0):
    wid = lax.axis_index("s") * NC + lax.axis_index("c")
    sems = (s0, s1)
    iota = lax.iota(jnp.int32, 16)
    iota_a = iota * APITCH

    def fire(t, pb):
        b = t * NW + wid
        for a in range(4):
            pltpu.async_copy(
                tabt_hbm.at[pl.ds(a * 8, 8), pl.ds(b * 128, 128)],
                stg.at[pb].at[a],
                sems[pb],
            )

    def drain(pb):
        for a in range(4):
            pltpu.make_async_copy(
                tabt_hbm.at[pl.ds(0, 8), pl.ds(0, 128)],
                stg.at[pb].at[a],
                sems[pb],
            ).wait()

    def transpose(pb):
        for a in range(4):
            def tbody(jm, carry, a=a):
                col = a * 8 + jm
                for p in range(8):
                    v = stg[pb, a, jm, pl.ds(p * 16, 16)]
                    plsc.store_scatter(tpa, [iota_a + (p * 16 * APITCH) + col], v)
                return carry

            lax.fori_loop(0, 8, tbody, 0)

        def qbody(q, carry):
            for c in range(4):
                for h in range(2):
                    tpb[q, pl.ds(c * 32 + h * 16, 16)] = tpa[
                        pl.ds((4 * q + c) * APITCH + h * 16, 16)
                    ]
            return carry

        lax.fori_loop(0, 32, qbody, 0)

    def write(t):
        b = t * NW + wid
        pltpu.async_copy(tpb, out_hbm.at[pl.ds(32 * b, 32)], w0)

    def wait_w():
        pltpu.make_async_copy(tpb, out_hbm.at[pl.ds(0, 32)], w0).wait()

    fire(0, 0)
    fire(1, 1)
    drain(0)
    transpose(0)
    write(0)
    fire(2, 0)

    def pair(i, carry):
        t1 = 2 * i + 1
        drain(1)
        wait_w()
        transpose(1)
        write(t1)

        @pl.when(t1 + 2 < TSLOT)
        def _():
            fire(t1 + 2, 1)

        t2 = 2 * i + 2
        drain(0)
        wait_w()
        transpose(0)
        write(t2)

        @pl.when(t2 + 2 < TSLOT)
        def _():
            fire(t2 + 2, 0)

        return carry

    lax.fori_loop(0, (TSLOT - 2) // 2, pair, 0)

    drain(1)
    wait_w()
    transpose(1)
    write(TSLOT - 1)

    @pl.when(wid < TBLK - TSLOT * NW)
    def _():
        fire(TSLOT, 0)
        drain(0)
        wait_w()
        transpose(0)
        write(TSLOT)

    @pl.when(wid == 4)
    def _():
        pltpu.sync_copy(tail_hbm, tailv)
        pltpu.async_copy(tailv, out_hbm.at[pl.ds(32 * TBLK, 16)], w0)
        pltpu.make_async_copy(
            tailv, out_hbm.at[pl.ds(0, 16)], w0
        ).wait()

    wait_w()


@functools.partial(
    pl.kernel,
    mesh=_mesh,
    out_type=jax.ShapeDtypeStruct(
        (FIELDS, EMBED_DIM // 8, BATCH // 128, 8, 128), jnp.float32
    ),
    scratch_types=[
        pltpu.VMEM((PERW,), jnp.int32),
        pltpu.VMEM((PERW,), jnp.int32),
        pltpu.VMEM((PERW,), jnp.int32),
        pltpu.VMEM((2, GW, 128), jnp.float32),
        pltpu.VMEM((EMBED_DIM, PITCH), jnp.float32),
        pltpu.SemaphoreType.DMA,
        pltpu.SemaphoreType.DMA,
        pltpu.SemaphoreType.DMA,
    ],
    compiler_params=pltpu.CompilerParams(
        use_tc_tiling_on_sc=True, needs_layout_passes=False
    ),
)
def _gather_kernel(idx_hbm, table_hbm, out_hbm, idx_v, blk_v, off_v, buf, tbuf,
                   g0, g1, w0):
    wid = lax.axis_index("s") * NC + lax.axis_index("c")
    pltpu.sync_copy(idx_hbm.at[pl.ds(wid * PERW, PERW)], idx_v)
    gsems = (g0, g1)
    iota = lax.iota(jnp.int32, 16)

    def prep(i, carry):
        v = idx_v[pl.ds(i * 16, 16)]
        blk_v[pl.ds(i * 16, 16)] = lax.shift_right_logical(v, 2)
        off_v[pl.ds(i * 16, 16)] = (v & 3) * 32
        return carry

    lax.fori_loop(0, PERW // 16, prep, 0)

    def fire(m, pb):
        for cc in range(G):
            pltpu.async_copy(
                table_hbm.at[blk_v.at[pl.ds((m * G + cc) * CHUNK, CHUNK)]],
                buf.at[pb].at[pl.ds(cc * CHUNK, CHUNK)],
                gsems[pb],
            )

    def drain_g(pb):
        for cc in range(G):
            pltpu.make_async_copy(
                table_hbm.at[pl.ds(0, CHUNK)],
                buf.at[pb].at[pl.ds(cc * CHUNK, CHUNK)],
                gsems[pb],
            ).wait()

    def transpose(m, pb):
        def body(t, carry):
            offs = off_v[pl.ds(m * GW + t * 16, 16)]
            for k in range(16):
                n = t * 16 + k
                off = offs[k]
                v0 = plsc.load_gather(buf.at[pb], [_full(n), iota + off])
                v1 = plsc.load_gather(buf.at[pb], [_full(n), iota + (off + 16)])
                plsc.store_scatter(tbuf, [iota, _full(n)], v0)
                plsc.store_scatter(tbuf, [iota + 16, _full(n)], v1)
            return carry

        lax.fori_loop(0, GW // 16, body, 0)

    def write(m):
        g_first = wid * J + m * G
        f = g_first // (BATCH // CHUNK)
        bb0 = g_first % (BATCH // CHUNK)
        for jb in range(EMBED_DIM // 8):
            for cc in range(G):
                pltpu.async_copy(
                    tbuf.at[pl.ds(jb * 8, 8), pl.ds(cc * CHUNK, CHUNK)],
                    out_hbm.at[f, jb, bb0 + cc],
                    w0,
                )

    def wait_w():
        for _ in range(EMBED_DIM // 8 * G):
            pltpu.make_async_copy(
                tbuf.at[pl.ds(0, 8), pl.ds(0, CHUNK)],
                out_hbm.at[0, 0, 0],
                w0,
            ).wait()

    fire(0, 0)
    fire(1, 1)
    drain_g(0)
    transpose(0, 0)
    write(0)
    fire(2, 0)

    def pair(i, carry):
        m1 = 2 * i + 1
        drain_g(1)
        wait_w()
        transpose(m1, 1)
        write(m1)
        fire(m1 + 2, 1)

        m2 = 2 * i + 2
        drain_g(0)
        wait_w()
        transpose(m2, 0)
        write(m2)

        @pl.when(i < (M - 4) // 2)
        def _():
            fire(m2 + 2, 0)

        return carry

    lax.fori_loop(0, (M - 2) // 2, pair, 0)

    drain_g(1)
    wait_w()
    transpose(M - 1, 1)
    write(M - 1)
    wait_w()


def kernel(bin_ids, table):
    tail = table[NROW - 64:].reshape(16, 128)
    tabl = _convert_kernel(jnp.swapaxes(table, 0, 1), tail)
    idx = jnp.swapaxes(bin_ids, 0, 1).reshape(-1)
    out5 = _gather_kernel(idx, tabl)
    return out5.transpose(2, 4, 0, 1, 3).reshape(BATCH, FIELDS, EMBED_DIM)


# speedup vs baseline: 1.3563x; 1.3563x over previous
import functools

import jax
import jax.numpy as jnp
from jax import lax
from jax.experimental import pallas as pl
from jax.experimental.pallas import tpu as pltpu
from jax.experimental.pallas import tpu_sc as plsc

BATCH = 16384
FIELDS = 26
EMBED_DIM = 32
B = BATCH * FIELDS
NC, NS = 2, 16
NW = NC * NS
CHUNK = 128
J = B // (NW * CHUNK)
G = 8
M = J // G
PERW = J * CHUNK
GW = G * CHUNK
PITCH = GW + 1

_mesh = plsc.VectorSubcoreMesh(core_axis_name="c", subcore_axis_name="s")


def _full(v):
    return jnp.full((16,), v, dtype=jnp.int32)


@functools.partial(
    pl.kernel,
    mesh=_mesh,
    out_type=jax.ShapeDtypeStruct(
        (FIELDS, EMBED_DIM // 8, BATCH // 128, 8, 128), jnp.float32
    ),
    scratch_types=[
        pltpu.VMEM((PERW,), jnp.int32),
        pltpu.VMEM((2, GW, EMBED_DIM), jnp.float32),
        pltpu.VMEM((EMBED_DIM, PITCH), jnp.float32),
        pltpu.SemaphoreType.DMA,
        pltpu.SemaphoreType.DMA,
        pltpu.SemaphoreType.DMA,
    ],
    compiler_params=pltpu.CompilerParams(
        use_tc_tiling_on_sc=False, needs_layout_passes=False
    ),
)
def _gather_kernel(idx_hbm, table_hbm, out_hbm, idx_v, buf, tbuf, g0, g1, w0):
    wid = lax.axis_index("s") * NC + lax.axis_index("c")
    pltpu.sync_copy(idx_hbm.at[pl.ds(wid * PERW, PERW)], idx_v)
    gsems = (g0, g1)
    iota = lax.iota(jnp.int32, 16)

    def fire(m, pb):
        for cc in range(G):
            pltpu.async_copy(
                table_hbm.at[idx_v.at[pl.ds((m * G + cc) * CHUNK, CHUNK)]],
                buf.at[pb].at[pl.ds(cc * CHUNK, CHUNK)],
                gsems[pb],
            )

    def drain_g(pb):
        for cc in range(G):
            pltpu.make_async_copy(
                table_hbm.at[pl.ds(0, CHUNK)],
                buf.at[pb].at[pl.ds(cc * CHUNK, CHUNK)],
                gsems[pb],
            ).wait()

    def transpose(pb):
        def body(n, carry):
            v0 = buf[pb, n, pl.ds(0, 16)]
            v1 = buf[pb, n, pl.ds(16, 16)]
            plsc.store_scatter(tbuf, [iota, _full(n)], v0)
            plsc.store_scatter(tbuf, [iota + 16, _full(n)], v1)
            return carry

        lax.fori_loop(0, GW, body, 0)

    def write(m):
        g_first = wid * J + m * G
        f = g_first // (BATCH // CHUNK)
        bb0 = g_first % (BATCH // CHUNK)
        for jb in range(EMBED_DIM // 8):
            for cc in range(G):
                pltpu.async_copy(
                    tbuf.at[pl.ds(jb * 8, 8), pl.ds(cc * CHUNK, CHUNK)],
                    out_hbm.at[f, jb, bb0 + cc],
                    w0,
                )

    def wait_w():
        for _ in range(EMBED_DIM // 8 * G):
            pltpu.make_async_copy(
                tbuf.at[pl.ds(0, 8), pl.ds(0, CHUNK)],
                out_hbm.at[0, 0, 0],
                w0,
            ).wait()

    fire(0, 0)
    fire(1, 1)
    drain_g(0)
    transpose(0)
    write(0)
    fire(2, 0)

    def pair(i, carry):
        m1 = 2 * i + 1
        drain_g(1)
        wait_w()
        transpose(1)
        write(m1)

        @pl.when(i < (M - 3) // 2)
        def _():
            fire(m1 + 2, 1)

        m2 = 2 * i + 2
        drain_g(0)
        wait_w()
        transpose(0)
        write(m2)

        @pl.when(i < (M - 3) // 2)
        def _():
            fire(m2 + 2, 0)

        return carry

    lax.fori_loop(0, (M - 1) // 2, pair, 0)
    wait_w()


def kernel(bin_ids, table):
    idx = jnp.swapaxes(bin_ids, 0, 1).reshape(-1)
    out5 = _gather_kernel(idx, table)
    return out5.transpose(2, 4, 0, 1, 3).reshape(BATCH, FIELDS, EMBED_DIM)
